# baseline (speedup 1.0000x reference)
import jax
import jax.numpy as jnp
from jax import lax
from jax.experimental import pallas as pl
from jax.experimental.pallas import tpu as pltpu
from jax.experimental.pallas import tpu_sc as plsc

NUM_FIELDS = 26
VOCAB = 100000
DIM = 64
BATCH = 16384
R = NUM_FIELDS * DIM

_NC, _NS = 2, 16
NW = _NC * _NS
ROWS_PER_W = R // NW
VEC = 16
NQ = 4
VQ = 25088
QSIZES = (VQ, VQ, VQ, 99968 - 3 * VQ)
TAIL0 = 99968
TAILW = VOCAB - TAIL0
NVEC_PART = BATCH // VEC


def _body(p_hbm, idx_hbm, out_hbm, qbuf, idxraw, pos, outbuf, tails,
          qsem0, qsem1, osem):
    wid = lax.axis_index("s") * _NC + lax.axis_index("c")
    r0 = wid * ROWS_PER_W
    lanes = lax.iota(jnp.int32, VEC)
    qsems = (qsem0, qsem1)

    def issue_quarter(r, q, slot):
        pltpu.async_copy(
            p_hbm.at[r, pl.ds(q * VQ, QSIZES[q])],
            qbuf.at[pl.ds(slot * VQ, QSIZES[q])],
            qsems[slot],
        )

    def wait_quarter(r, q, slot):
        pltpu.make_async_copy(
            p_hbm.at[r, pl.ds(q * VQ, QSIZES[q])],
            qbuf.at[pl.ds(slot * VQ, QSIZES[q])],
            qsems[slot],
        ).wait()

    def partition(f):
        pltpu.sync_copy(idx_hbm.at[f], idxraw)
        pltpu.sync_copy(
            p_hbm.at[pl.ds(f * DIM, DIM), pl.ds(TAIL0, TAILW)], tails
        )
        bounds = []
        w = jnp.zeros((VEC,), jnp.int32)
        for q in range(NQ + 1):
            lo = q * VQ

            def pstep(t, w):
                js = [t * 4 + u for u in range(4)]
                vs = [idxraw[pl.ds(j * VEC, VEC)] for j in js]
                if q == 0:
                    ms = [v < VQ for v in vs]
                elif q < NQ:
                    hi = min(lo + VQ, TAIL0)
                    ms = [jnp.logical_and(v >= lo, v < hi) for v in vs]
                else:
                    ms = [v >= TAIL0 for v in vs]
                cs = [plsc.cumsum(m.astype(jnp.int32)) for m in ms]
                ps = [plsc.all_reduce_population_count(m) for m in ms]
                for u in range(4):
                    plsc.store_scatter(
                        pos, [w + cs[u] - 1], lanes + js[u] * VEC, mask=ms[u]
                    )
                    w = w + ps[u]
                return w

            w = lax.fori_loop(0, NVEC_PART // 4, pstep, w)
            if q < NQ:
                bounds.append(w[0])
        return bounds[0], bounds[1], bounds[2], bounds[3]

    GU = 4

    def masked_vec(j, s_lo, s_hi, base, src, out_off):
        k = j * VEC
        pvec = pos[pl.ds(k, VEC)]
        kv = lanes + k
        m = jnp.logical_and(kv >= s_lo, kv < s_hi)
        vvec = plsc.load_gather(idxraw, [pvec], mask=m)
        g = plsc.load_gather(src, [vvec + base], mask=m)
        plsc.store_scatter(outbuf, [pvec + out_off], g, mask=m)

    def gather_span(s_lo, s_hi, base, src, out_off):
        a = s_lo // VEC
        b = s_hi // VEC
        masked_vec(a, s_lo, s_hi, base, src, out_off)

        @pl.when(b > a)
        def _():
            masked_vec(b, s_lo, s_hi, base, src, out_off)

        def gstep(t, _):
            j0 = a + 1 + t * GU
            ks = [(j0 + u) * VEC for u in range(GU)]
            pvecs = [pos[pl.ds(k, VEC)] for k in ks]
            vvecs = [plsc.load_gather(idxraw, [pv]) for pv in pvecs]
            gs = [plsc.load_gather(src, [vv + base]) for vv in vvecs]
            for pv, g in zip(pvecs, gs):
                plsc.store_scatter(outbuf, [pv + out_off], g)
            return 0

        n_int = jnp.maximum(b - (a + 1), 0)
        n_grp = n_int // GU
        lax.fori_loop(0, n_grp, gstep, 0)

        def rstep(t, _):
            masked_vec(a + 1 + n_grp * GU + t, s_lo, s_hi, base,
                       src, out_off)
            return 0

        lax.fori_loop(0, n_int - n_grp * GU, rstep, 0)

    issue_quarter(r0, 0, 0)

    def do_row(i, carry):
        prev_f, s1, s2, s3, s4 = carry
        r = r0 + i
        f = r // DIM

        s1, s2, s3, s4 = lax.cond(
            f != prev_f,
            lambda: partition(f),
            lambda: (s1, s2, s3, s4),
        )

        out_off = (i % 2) * BATCH
        starts = (jnp.int32(0), s1, s2, s3, s4)
        for q in range(NQ):
            slot = q % 2
            if q < NQ - 1:
                issue_quarter(r, q + 1, (q + 1) % 2)
            else:
                @pl.when(i + 1 < ROWS_PER_W)
                def _():
                    issue_quarter(r + 1, 0, 0)
            wait_quarter(r, q, slot)
            gather_span(
                starts[q], starts[q + 1], slot * VQ - q * VQ, qbuf, out_off
            )

        trow = jnp.full((VEC,), r - f * DIM, jnp.int32)

        def tstep(j, _):
            k = j * VEC
            pvec = pos[pl.ds(k, VEC)]
            kv = lanes + k
            m = kv >= s4
            vvec = plsc.load_gather(idxraw, [pvec], mask=m)
            g = plsc.load_gather(tails, [trow, vvec - TAIL0], mask=m)
            plsc.store_scatter(outbuf, [pvec + out_off], g, mask=m)
            return 0

        lax.fori_loop(s4 // VEC, BATCH // VEC, tstep, 0)

        pltpu.sync_copy(outbuf.at[pl.ds(out_off, BATCH)], out_hbm.at[r])
        return f, s1, s2, s3, s4

    lax.fori_loop(
        0, ROWS_PER_W, do_row,
        (jnp.int32(-1), jnp.int32(0), jnp.int32(0), jnp.int32(0),
         jnp.int32(0)),
    )


def kernel(indices, tables):
    p = jnp.transpose(tables, (0, 2, 1)).reshape(R, VOCAB)
    idx_t = jnp.transpose(indices.astype(jnp.int32), (1, 0))

    mesh = plsc.VectorSubcoreMesh(core_axis_name="c", subcore_axis_name="s")
    out_t = pl.kernel(
        _body,
        out_type=jax.ShapeDtypeStruct((R, BATCH), jnp.float32),
        mesh=mesh,
        scratch_types=[
            pltpu.VMEM((2 * VQ,), jnp.float32),
            pltpu.VMEM((BATCH,), jnp.int32),
            pltpu.VMEM((BATCH + 4 * VEC,), jnp.int32),
            pltpu.VMEM((2 * BATCH,), jnp.float32),
            pltpu.VMEM((DIM, TAILW), jnp.float32),
            pltpu.SemaphoreType.DMA,
            pltpu.SemaphoreType.DMA,
            pltpu.SemaphoreType.DMA,
        ],
        compiler_params=pltpu.CompilerParams(needs_layout_passes=False),
    )(p, idx_t)
    return out_t.T.reshape(BATCH, NUM_FIELDS * DIM)


# speedup vs baseline: 1.0625x; 1.0625x over previous
import jax
import jax.numpy as jnp
from jax import lax
from jax.experimental import pallas as pl
from jax.experimental.pallas import tpu as pltpu
from jax.experimental.pallas import tpu_sc as plsc

NUM_FIELDS = 26
VOCAB = 100000
DIM = 64
BATCH = 16384
R = NUM_FIELDS * DIM

_NC, _NS = 2, 16
NW = _NC * _NS
ROWS_PER_W = R // NW
OUT_CHUNK = BATCH // 2
VEC = 16
UNROLL = 8


def _body(p_hbm, idx_hbm, out_hbm, row_v, idx_v, out_v, sem):
    wid = lax.axis_index("s") * _NC + lax.axis_index("c")
    r0 = wid * ROWS_PER_W
    n_chunk = BATCH // OUT_CHUNK

    def do_row(i, prev_f):
        r = r0 + i
        f = r // DIM

        @pl.when(jnp.logical_or(i == 0, f != prev_f))
        def _():
            pltpu.sync_copy(idx_hbm.at[f], idx_v)

        pltpu.sync_copy(p_hbm.at[r], row_v)

        for h in range(n_chunk):
            def gather_step(j, _):
                base = h * OUT_CHUNK + j * (VEC * UNROLL)
                ivs = [idx_v[pl.ds(base + u * VEC, VEC)] for u in range(UNROLL)]
                gs = [plsc.load_gather(row_v, [iv]) for iv in ivs]
                for u in range(UNROLL):
                    out_v[pl.ds(j * (VEC * UNROLL) + u * VEC, VEC)] = gs[u]
                return 0

            lax.fori_loop(0, OUT_CHUNK // (VEC * UNROLL), gather_step, 0)
            pltpu.sync_copy(out_v, out_hbm.at[r, pl.ds(h * OUT_CHUNK, OUT_CHUNK)])
        return f

    lax.fori_loop(0, ROWS_PER_W, do_row, jnp.int32(-1))


def kernel(indices, tables):
    p = jnp.transpose(tables, (0, 2, 1)).reshape(R, VOCAB)
    idx_t = jnp.transpose(indices.astype(jnp.int32), (1, 0))

    mesh = plsc.VectorSubcoreMesh(core_axis_name="c", subcore_axis_name="s")
    out_t = pl.kernel(
        _body,
        out_type=jax.ShapeDtypeStruct((R, BATCH), jnp.float32),
        mesh=mesh,
        scratch_types=[
            pltpu.VMEM((VOCAB,), jnp.float32),
            pltpu.VMEM((BATCH,), jnp.int32),
            pltpu.VMEM((OUT_CHUNK,), jnp.float32),
            pltpu.SemaphoreType.DMA,
        ],
        compiler_params=pltpu.CompilerParams(needs_layout_passes=False),
    )(p, idx_t)
    return out_t.T.reshape(BATCH, NUM_FIELDS * DIM)
